# Initial kernel scaffold; baseline (speedup 1.0000x reference)
import functools

import jax
import jax.numpy as jnp
from jax import lax
from jax.experimental import pallas as pl
from jax.experimental.pallas import tpu as pltpu
from jax.experimental.pallas import tpu_sc as plsc

_STREAM = 128


def _make_gather(n, V, D, NC, NS):
    NW = NC * NS
    b_per_w = n // NW
    CH = 1024
    K = CH // _STREAM
    n_chunks = b_per_w // CH

    mesh = plsc.VectorSubcoreMesh(core_axis_name="c", subcore_axis_name="s")

    @functools.partial(
        pl.kernel,
        mesh=mesh,
        out_type=jax.ShapeDtypeStruct((n, D), jnp.float32),
        scratch_types=[
            pltpu.VMEM((b_per_w,), jnp.int32),
            pltpu.VMEM((CH, D), jnp.float32),
            pltpu.SemaphoreType.DMA,
        ],
    )
    def gather(idx_hbm, table_hbm, out_hbm, idx_v, rows_v, sem):
        wid = lax.axis_index("s") * NC + lax.axis_index("c")
        base = pl.multiple_of(wid * b_per_w, CH)
        pltpu.sync_copy(idx_hbm.at[pl.ds(base, b_per_w)], idx_v)

        def body(g, carry):
            off = pl.multiple_of(g * CH, CH)
            for j in range(K):
                jo = pl.multiple_of(off + j * _STREAM, _STREAM)
                pltpu.async_copy(
                    table_hbm.at[idx_v.at[pl.ds(jo, _STREAM)]],
                    rows_v.at[pl.ds(j * _STREAM, _STREAM)],
                    sem,
                ).wait()
            pltpu.sync_copy(rows_v, out_hbm.at[pl.ds(base + off, CH)])
            return carry

        lax.fori_loop(0, n_chunks, body, 0)

    return gather


def kernel(x, table):
    B, H = x.shape
    V, D = table.shape
    n = B * H
    idx = x.reshape(n).astype(jnp.int32)
    info = plsc.get_sparse_core_info()
    gather = _make_gather(n, V, D, info.num_cores, info.num_subcores)
    out = gather(idx, table)
    return out.reshape(B, H, D)


# speedup vs baseline: 1.0370x; 1.0370x over previous
import functools

import jax
import jax.numpy as jnp
from jax import lax
from jax.experimental import pallas as pl
from jax.experimental.pallas import tpu as pltpu
from jax.experimental.pallas import tpu_sc as plsc

_STREAM = 128


def _make_gather(n, V, D, NC, NS):
    NW = NC * NS
    b_per_w = n // NW
    CH = 1024
    K = CH // _STREAM
    n_chunks = b_per_w // CH

    mesh = plsc.VectorSubcoreMesh(core_axis_name="c", subcore_axis_name="s")

    @functools.partial(
        pl.kernel,
        mesh=mesh,
        compiler_params=pltpu.CompilerParams(use_tc_tiling_on_sc=False),
        out_type=jax.ShapeDtypeStruct((n, D), jnp.float32),
        scratch_types=[
            pltpu.VMEM((b_per_w,), jnp.int32),
            pltpu.VMEM((CH, D), jnp.float32),
            pltpu.SemaphoreType.DMA,
        ],
    )
    def gather(idx_hbm, table_hbm, out_hbm, idx_v, rows_v, sem):
        wid = lax.axis_index("s") * NC + lax.axis_index("c")
        base = pl.multiple_of(wid * b_per_w, CH)
        pltpu.sync_copy(idx_hbm.at[pl.ds(base, b_per_w)], idx_v)

        def body(g, carry):
            off = pl.multiple_of(g * CH, CH)
            for j in range(K):
                jo = pl.multiple_of(off + j * _STREAM, _STREAM)
                pltpu.async_copy(
                    table_hbm.at[idx_v.at[pl.ds(jo, _STREAM)]],
                    rows_v.at[pl.ds(j * _STREAM, _STREAM)],
                    sem,
                ).wait()
            pltpu.sync_copy(rows_v, out_hbm.at[pl.ds(base + off, CH)])
            return carry

        lax.fori_loop(0, n_chunks, body, 0)

    return gather


def kernel(x, table):
    B, H = x.shape
    V, D = table.shape
    n = B * H
    idx = x.reshape(n).astype(jnp.int32)
    info = plsc.get_sparse_core_info()
    gather = _make_gather(n, V, D, info.num_cores, info.num_subcores)
    out = gather(idx, table)
    return out.reshape(B, H, D)


# speedup vs baseline: 1.1119x; 1.0722x over previous
import functools

import jax
import jax.numpy as jnp
from jax import lax
from jax.experimental import pallas as pl
from jax.experimental.pallas import tpu as pltpu
from jax.experimental.pallas import tpu_sc as plsc

_STREAM = 128


def _make_gather(n, V, D, NC, NS):
    NW = NC * NS
    b_per_w = n // NW
    CH = 1280
    K = CH // _STREAM
    n_chunks = b_per_w // CH
    n_pairs = n_chunks // 2

    mesh = plsc.VectorSubcoreMesh(core_axis_name="c", subcore_axis_name="s")

    @functools.partial(
        pl.kernel,
        mesh=mesh,
        compiler_params=pltpu.CompilerParams(use_tc_tiling_on_sc=False),
        out_type=jax.ShapeDtypeStruct((n, D), jnp.float32),
        scratch_types=[
            pltpu.VMEM((b_per_w,), jnp.int32),
            pltpu.VMEM((2, CH, D), jnp.float32),
            pltpu.SemaphoreType.DMA,
            pltpu.SemaphoreType.DMA,
            pltpu.SemaphoreType.DMA,
            pltpu.SemaphoreType.DMA,
        ],
    )
    def gather(idx_hbm, table_hbm, out_hbm, idx_v, rows_v, g0, g1, s0, s1):
        gsem = (g0, g1)
        ssem = (s0, s1)
        wid = lax.axis_index("s") * NC + lax.axis_index("c")
        base = pl.multiple_of(wid * b_per_w, CH)
        pltpu.sync_copy(idx_hbm.at[pl.ds(base, b_per_w)], idx_v)

        def fire_gathers(g, b):
            off = pl.multiple_of(g * CH, CH)
            for j in range(K):
                jo = pl.multiple_of(off + j * _STREAM, _STREAM)
                pltpu.async_copy(
                    table_hbm.at[idx_v.at[pl.ds(jo, _STREAM)]],
                    rows_v.at[b].at[pl.ds(j * _STREAM, _STREAM)],
                    gsem[b],
                )

        def wait_gathers(b):
            pltpu.make_async_copy(
                out_hbm.at[pl.ds(0, CH)], rows_v.at[b], gsem[b]
            ).wait()

        def fire_store(g, b):
            off = pl.multiple_of(g * CH, CH)
            pltpu.async_copy(rows_v.at[b], out_hbm.at[pl.ds(base + off, CH)], ssem[b])

        def wait_store(b):
            pltpu.make_async_copy(
                rows_v.at[b], out_hbm.at[pl.ds(base, CH)], ssem[b]
            ).wait()

        fire_gathers(0, 0)

        def pair(i, carry):
            g = pl.multiple_of(i * 2, 2)
            wait_gathers(0)

            @pl.when(i > 0)
            def _():
                wait_store(1)

            fire_gathers(g + 1, 1)
            fire_store(g, 0)

            wait_gathers(1)
            wait_store(0)

            @pl.when(i < n_pairs - 1)
            def _():
                fire_gathers(g + 2, 0)

            fire_store(g + 1, 1)
            return carry

        lax.fori_loop(0, n_pairs, pair, 0)
        wait_store(1)

    return gather


def kernel(x, table):
    B, H = x.shape
    V, D = table.shape
    n = B * H
    idx = x.reshape(n).astype(jnp.int32)
    info = plsc.get_sparse_core_info()
    gather = _make_gather(n, V, D, info.num_cores, info.num_subcores)
    out = gather(idx, table)
    return out.reshape(B, H, D)


# speedup vs baseline: 1.7906x; 1.6103x over previous
import functools

import jax
import jax.numpy as jnp
from jax import lax
from jax.experimental import pallas as pl
from jax.experimental.pallas import tpu as pltpu
from jax.experimental.pallas import tpu_sc as plsc


def _make_gather(B, H, V, D, NC, NS):
    NW = NC * NS
    r_per_w = B // NW
    CH = 16
    n_chunks = r_per_w // CH
    n_pairs = n_chunks // 2

    mesh = plsc.VectorSubcoreMesh(core_axis_name="c", subcore_axis_name="s")

    @functools.partial(
        pl.kernel,
        mesh=mesh,
        compiler_params=pltpu.CompilerParams(use_tc_tiling_on_sc=False),
        out_type=jax.ShapeDtypeStruct((B, H, D), jnp.float32),
        scratch_types=[
            pltpu.VMEM((r_per_w, H), jnp.int32),
            pltpu.VMEM((2, CH, H, D), jnp.float32),
            pltpu.SemaphoreType.DMA,
            pltpu.SemaphoreType.DMA,
            pltpu.SemaphoreType.DMA,
            pltpu.SemaphoreType.DMA,
        ],
    )
    def gather(idx_hbm, table_hbm, out_hbm, idx_v, rows_v, g0, g1, s0, s1):
        gsem = (g0, g1)
        ssem = (s0, s1)
        wid = lax.axis_index("s") * NC + lax.axis_index("c")
        base = pl.multiple_of(wid * r_per_w, CH)
        pltpu.sync_copy(idx_hbm.at[pl.ds(base, r_per_w)], idx_v)

        def fire_gathers(g, b):
            off = pl.multiple_of(g * CH, CH)
            for j in range(CH):
                pltpu.async_copy(
                    table_hbm.at[idx_v.at[off + j]],
                    rows_v.at[b].at[j],
                    gsem[b],
                )

        def wait_gathers(b):
            pltpu.make_async_copy(
                out_hbm.at[pl.ds(0, CH)], rows_v.at[b], gsem[b]
            ).wait()

        def fire_store(g, b):
            off = pl.multiple_of(g * CH, CH)
            pltpu.async_copy(rows_v.at[b], out_hbm.at[pl.ds(base + off, CH)], ssem[b])

        def wait_store(b):
            pltpu.make_async_copy(
                rows_v.at[b], out_hbm.at[pl.ds(base, CH)], ssem[b]
            ).wait()

        fire_gathers(0, 0)

        def pair(i, carry):
            g = pl.multiple_of(i * 2, 2)
            wait_gathers(0)

            @pl.when(i > 0)
            def _():
                wait_store(1)

            fire_gathers(g + 1, 1)
            fire_store(g, 0)

            wait_gathers(1)
            wait_store(0)

            @pl.when(i < n_pairs - 1)
            def _():
                fire_gathers(g + 2, 0)

            fire_store(g + 1, 1)
            return carry

        lax.fori_loop(0, n_pairs, pair, 0)
        wait_store(1)

    return gather


def kernel(x, table):
    B, H = x.shape
    V, D = table.shape
    idx = x.astype(jnp.int32)
    info = plsc.get_sparse_core_info()
    gather = _make_gather(B, H, V, D, info.num_cores, info.num_subcores)
    return gather(idx, table)
